# Initial kernel scaffold; baseline (speedup 1.0000x reference)
import jax
import jax.numpy as jnp
from jax.experimental import pallas as pl


def kernel(x, edge_index, batch, edge_attr, edge_emb, W1, We1, as1, ad1, ae1, b1, W2, We2, as2, ad2, ae2, b2, W3, We3, as3, ad3, ae3, b3, W4, We4, as4, ad4, ae4, b4):
    raise NotImplementedError("write your pallas kernel here")


# speedup vs baseline: 22.0300x; 22.0300x over previous
import functools

import jax
import jax.numpy as jnp
import numpy as np
from jax import lax
from jax.experimental import pallas as pl
from jax.experimental.pallas import tpu as pltpu
from jax.experimental.pallas import tpu_sc as plsc

N = 10000
E = 320000
D = 128
ED = 32
T = 16

NC = 2
NS = 16
NT = NC * NS
CH = 128
B = 128
SB = 1024
NSB = 10
NJ = SB // B
TILE_E = NSB * SB
TROWS = TILE_E // CH
EPAD = NT * TILE_E
EROWS = EPAD // CH
NROW = N // NS

_mesh = plsc.VectorSubcoreMesh(core_axis_name="c", subcore_axis_name="s")
_sc_params = pltpu.CompilerParams(use_tc_tiling_on_sc=False,
                                  needs_layout_passes=False)


def _full(v):
    return jnp.full((16,), v, jnp.int32)


def _zero_acc(acc, buf, sid, width):
    zero = jnp.zeros((16,), jnp.float32)
    nv = width // 16

    def zrow(r, _):
        for c in range(nv):
            buf[r, pl.ds(c * 16, 16)] = zero
        return 0
    lax.fori_loop(0, B, zrow, 0)
    base = sid * NROW
    for off in range(0, NROW, B):
        sz = min(B, NROW - off)
        pltpu.sync_copy(buf.at[pl.ds(0, sz)], acc.at[pl.ds(base + off, sz)])


def _sc_cnt_body(dst2, attr1, out, acc, buf, dstb, attrb):
    cid = lax.axis_index("c")
    sid = lax.axis_index("s")
    tid = cid * NS + sid
    zero = jnp.zeros((16,), jnp.float32)
    one = jnp.ones((16,), jnp.float32)
    iota = lax.iota(jnp.int32, 16)

    _zero_acc(acc, buf, sid, T)
    plsc.subcore_barrier()

    def superbatch(sb, _):
        pltpu.sync_copy(dst2.at[pl.ds(tid * TROWS + sb * NJ, NJ)], dstb)
        pltpu.sync_copy(attr1.at[pl.ds(tid * TILE_E + sb * SB, SB)], attrb)

        def sub(j, _):
            ebase = tid * TILE_E + sb * SB + j * B

            def zrow2(r, _):
                buf[r, :] = zero
                return 0
            lax.fori_loop(0, B, zrow2, 0)

            def grp(g, _):
                ev = g * 16 + iota
                attrv = plsc.load_gather(attrb, [j * B + ev])
                valid = (ebase + ev) < E
                plsc.store_scatter(buf, [ev, attrv], one, mask=valid)
                return 0
            lax.fori_loop(0, B // 16, grp, 0)
            pltpu.sync_copy(buf, acc.at[dstb.at[j]], add=True)
            return 0
        lax.fori_loop(0, NJ, sub, 0)
        return 0
    lax.fori_loop(0, NSB, superbatch, 0)
    plsc.subcore_barrier()

    @pl.when(sid == 0)
    def _():
        pltpu.sync_copy(acc, out.at[pl.ds(cid * N, N)])


_sc_cnt = functools.partial(
    pl.kernel,
    out_type=jax.ShapeDtypeStruct((NC * N, T), jnp.float32),
    mesh=_mesh,
    compiler_params=_sc_params,
    scratch_types=[
        pltpu.VMEM_SHARED((N, T), jnp.float32),
        pltpu.VMEM((B, T), jnp.float32),
        pltpu.VMEM((NJ, CH), jnp.int32),
        pltpu.VMEM((SB,), jnp.int32),
    ],
)(_sc_cnt_body)


def _sc_layer_body(h, xcat, aldt, tbl16, src2, dst2, attr1, out,
                   acc, buf, aldbuf, pbuf, srcb, dstb, attrb, tblb,
                   semg, sema):
    cid = lax.axis_index("c")
    sid = lax.axis_index("s")
    tid = cid * NS + sid
    zero = jnp.zeros((16,), jnp.float32)
    iota = lax.iota(jnp.int32, 16)

    pltpu.sync_copy(tbl16, tblb)

    def zp(r, _):
        pbuf[r, :] = zero
        return 0
    lax.fori_loop(0, B, zp, 0)

    _zero_acc(acc, buf, sid, 144)
    plsc.subcore_barrier()

    def superbatch(sb, _):
        pltpu.sync_copy(src2.at[pl.ds(tid * TROWS + sb * NJ, NJ)], srcb)
        pltpu.sync_copy(dst2.at[pl.ds(tid * TROWS + sb * NJ, NJ)], dstb)
        pltpu.sync_copy(attr1.at[pl.ds(tid * TILE_E + sb * SB, SB)], attrb)

        def sub(j, _):
            ebase = tid * TILE_E + sb * SB + j * B
            dg = pltpu.async_copy(xcat.at[srcb.at[j]], buf, semg)
            da = pltpu.async_copy(aldt.at[dstb.at[j]], aldbuf, sema)
            dg.wait()
            da.wait()

            def passA(g, _):
                ev = g * 16 + iota
                attrv = plsc.load_gather(attrb, [j * B + ev])
                valid = (ebase + ev) < E
                for hh in range(h):
                    alsv = plsc.load_gather(buf, [ev, _full(128 + hh)])
                    aldv = plsc.load_gather(aldbuf, [ev, _full(hh)])
                    tblv = plsc.load_gather(tblb, [attrv, _full(hh)])
                    s = alsv + aldv + tblv
                    s = jnp.where(s > 0, s, 0.2 * s)
                    p = jnp.where(valid, jnp.exp(s), 0.0)
                    plsc.store_scatter(pbuf, [ev, _full(hh)], p)
                return 0
            lax.fori_loop(0, B // 16, passA, 0)

            def passB(g, _):
                ev = g * 16 + iota
                for cg in range(8):
                    hh = cg if h == 8 else 0
                    pv = plsc.load_gather(pbuf, [ev, _full(hh)])
                    for o in range(16):
                        c = cg * 16 + o
                        v = plsc.load_gather(buf, [ev, _full(c)])
                        plsc.store_scatter(buf, [ev, _full(c)], v * pv)
                for o in range(16):
                    pv2 = plsc.load_gather(pbuf, [ev, _full(o)])
                    plsc.store_scatter(buf, [ev, _full(128 + o)], pv2)
                return 0
            lax.fori_loop(0, B // 16, passB, 0)

            pltpu.sync_copy(buf, acc.at[dstb.at[j]], add=True)
            return 0
        lax.fori_loop(0, NJ, sub, 0)
        return 0
    lax.fori_loop(0, NSB, superbatch, 0)
    plsc.subcore_barrier()

    @pl.when(sid == 0)
    def _():
        pltpu.sync_copy(acc, out.at[pl.ds(cid * N, N)])


def _make_sc_layer(h):
    return functools.partial(
        pl.kernel,
        out_type=jax.ShapeDtypeStruct((NC * N, 144), jnp.float32),
        mesh=_mesh,
        compiler_params=_sc_params,
        scratch_types=[
            pltpu.VMEM_SHARED((N, 144), jnp.float32),
            pltpu.VMEM((B, 144), jnp.float32),
            pltpu.VMEM((B, 16), jnp.float32),
            pltpu.VMEM((B, 16), jnp.float32),
            pltpu.VMEM((NJ, CH), jnp.int32),
            pltpu.VMEM((NJ, CH), jnp.int32),
            pltpu.VMEM((SB,), jnp.int32),
            pltpu.VMEM((T, 16), jnp.float32),
            pltpu.SemaphoreType.DMA,
            pltpu.SemaphoreType.DMA,
        ],
    )(functools.partial(_sc_layer_body, h))


_sc_layer8 = _make_sc_layer(8)
_sc_layer1 = _make_sc_layer(1)


R = 2000
G = N // R


def _prologue_body(c0, c1, emb, wa1, wa2, wa3, wa4, loope, t1, t2, t3, t4):
    i = pl.program_id(0)
    cntT = c0[...] + c1[...]
    cnt = jnp.sum(cntT, axis=1, keepdims=True)
    loope[...] = jnp.dot(cntT, emb[...], preferred_element_type=jnp.float32) \
        / jnp.maximum(cnt, 1.0)

    @pl.when(i == 0)
    def _():
        e = emb[...]
        t1[...] = jnp.dot(e, wa1[...], preferred_element_type=jnp.float32)
        t2[...] = jnp.dot(e, wa2[...], preferred_element_type=jnp.float32)
        t3[...] = jnp.dot(e, wa3[...], preferred_element_type=jnp.float32)
        t4[...] = jnp.dot(e, wa4[...], preferred_element_type=jnp.float32)


def _prologue(c0, c1, emb, wa1, wa2, wa3, wa4):
    fix = lambda i: (0, 0)
    blk = lambda i: (i, 0)
    return pl.pallas_call(
        _prologue_body,
        grid=(G,),
        in_specs=[pl.BlockSpec((R, T), blk), pl.BlockSpec((R, T), blk),
                  pl.BlockSpec((T, ED), fix)] +
                 [pl.BlockSpec((ED, 16), fix)] * 4,
        out_specs=[pl.BlockSpec((R, ED), blk)] + [pl.BlockSpec((T, 16), fix)] * 4,
        out_shape=[jax.ShapeDtypeStruct((N, ED), jnp.float32)] +
                  [jax.ShapeDtypeStruct((T, 16), jnp.float32)] * 4,
    )(c0, c1, emb, wa1, wa2, wa3, wa4)


def _prep_body(x, wcat, wd, xcat, aldt):
    xb = x[...]
    xcat[...] = jnp.dot(xb, wcat[...], preferred_element_type=jnp.float32)
    aldt[...] = jnp.dot(xb, wd[...], preferred_element_type=jnp.float32)


def _prep(xin, Wcat, Wd):
    din = xin.shape[1]
    fix = lambda i: (0, 0)
    blk = lambda i: (i, 0)
    return pl.pallas_call(
        _prep_body,
        grid=(G,),
        in_specs=[pl.BlockSpec((R, din), blk), pl.BlockSpec((din, 144), fix),
                  pl.BlockSpec((din, 16), fix)],
        out_specs=[pl.BlockSpec((R, 144), blk), pl.BlockSpec((R, 16), blk)],
        out_shape=[jax.ShapeDtypeStruct((N, 144), jnp.float32),
                   jax.ShapeDtypeStruct((N, 16), jnp.float32)],
    )(xin, Wcat, Wd)


def _finalize_body(h, relu, w0, w1, pz0, pz1, xs, als, aldt, loope, wea, bias,
                   xprev, xout):
    z = pz0[...] + pz1[...]
    aeloop = jnp.dot(loope[...], wea[...], preferred_element_type=jnp.float32)
    sl = als[...] + aldt[...] + aeloop
    ploop = jnp.exp(jnp.where(sl > 0, sl, 0.2 * sl))
    zt = z + ploop + 1e-16
    rep = 128 // h
    p128 = jnp.repeat(ploop[:, :h], rep, axis=1)
    z128 = jnp.repeat(zt[:, :h], rep, axis=1)
    y = (w0[...] + w1[...] + xs[...] * p128) / z128 + bias[...]
    if relu:
        y = jnp.maximum(y, 0.0)
    xout[...] = y + xprev[...]


def _finalize(h, relu, w0, w1, pz0, pz1, xs, als, aldt, loope, wea, bias, xprev):
    fix = lambda i: (0, 0)
    blk = lambda i: (i, 0)
    return pl.pallas_call(
        functools.partial(_finalize_body, h, relu),
        grid=(G,),
        in_specs=[pl.BlockSpec((R, 128), blk), pl.BlockSpec((R, 128), blk),
                  pl.BlockSpec((R, 16), blk), pl.BlockSpec((R, 16), blk),
                  pl.BlockSpec((R, 128), blk), pl.BlockSpec((R, 16), blk),
                  pl.BlockSpec((R, 16), blk), pl.BlockSpec((R, ED), blk),
                  pl.BlockSpec((ED, 16), fix), pl.BlockSpec((1, 128), fix),
                  pl.BlockSpec((R, 128), blk)],
        out_specs=pl.BlockSpec((R, 128), blk),
        out_shape=jax.ShapeDtypeStruct((N, 128), jnp.float32),
    )(w0, w1, pz0, pz1, xs, als, aldt, loope, wea, bias, xprev)


def _weight_prep(W, We, a_s, a_d, a_e, h, oc):
    din = W.shape[0]
    Wr = W.reshape(din, h, oc)
    ws = jnp.einsum('dho,ho->dh', Wr, a_s)
    wd = jnp.einsum('dho,ho->dh', Wr, a_d)
    wea = jnp.einsum('kho,ho->kh', We.reshape(ED, h, oc), a_e)
    zpad = jnp.zeros((din, 16 - h), jnp.float32)
    Wcat = jnp.concatenate([W, ws, zpad], axis=1)
    Wd16 = jnp.concatenate([wd, zpad], axis=1)
    Wea16 = jnp.concatenate([wea, jnp.zeros((ED, 16 - h), jnp.float32)], axis=1)
    return Wcat, Wd16, Wea16


def kernel(x, edge_index, batch, edge_attr, edge_emb,
           W1, We1, as1, ad1, ae1, b1,
           W2, We2, as2, ad2, ae2, b2,
           W3, We3, as3, ad3, ae3, b3,
           W4, We4, as4, ad4, ae4, b4):
    src = edge_index[0]
    dst = edge_index[1]
    pad = EPAD - E
    src2 = jnp.pad(src, (0, pad)).reshape(EROWS, CH)
    dst2 = jnp.pad(dst, (0, pad)).reshape(EROWS, CH)
    attr1 = jnp.pad(edge_attr, (0, pad))

    layers = [
        _weight_prep(W1, We1, as1, ad1, ae1, 8, 16) + (b1, 8, True),
        _weight_prep(W2, We2, as2, ad2, ae2, 8, 16) + (b2, 8, True),
        _weight_prep(W3, We3, as3, ad3, ae3, 8, 16) + (b3, 8, True),
        _weight_prep(W4, We4, as4, ad4, ae4, 1, 128) + (b4, 1, False),
    ]

    cntp = _sc_cnt(dst2, attr1)
    loope, tb1, tb2, tb3, tb4 = _prologue(
        cntp[:N], cntp[N:], edge_emb,
        layers[0][2], layers[1][2], layers[2][2], layers[3][2])
    tbls = [tb1, tb2, tb3, tb4]

    xcur = x
    for li, (Wcat, Wd16, Wea16, b, h, relu) in enumerate(layers):
        resid = jnp.zeros((N, 128), jnp.float32) if li == 0 else xcur
        xcat, aldt = _prep(xcur, Wcat, Wd16)
        accp = (_sc_layer8 if h == 8 else _sc_layer1)(
            xcat, aldt, tbls[li], src2, dst2, attr1)
        a0, a1 = accp[:N], accp[N:]
        xcur = _finalize(h, relu,
                         a0[:, :128], a1[:, :128], a0[:, 128:], a1[:, 128:],
                         xcat[:, :128], xcat[:, 128:], aldt, loope, Wea16,
                         b.reshape(1, 128), resid)
    return (xcur, batch)


# speedup vs baseline: 35.0070x; 1.5891x over previous
import functools

import jax
import jax.numpy as jnp
import numpy as np
from jax import lax
from jax.experimental import pallas as pl
from jax.experimental.pallas import tpu as pltpu
from jax.experimental.pallas import tpu_sc as plsc

N = 10000
E = 320000
D = 128
ED = 32
T = 16

NC = 2
NS = 16
NT = NC * NS
CH = 128
B = 128
SB = 1024
NSB = 10
NJ = SB // B
TILE_E = NSB * SB
TROWS = TILE_E // CH
EPAD = NT * TILE_E
EROWS = EPAD // CH
NROW = N // NS

W80 = 80
TILE_E2 = EPAD // NS
TROWS2 = TILE_E2 // CH
NSB2 = TILE_E2 // SB
NBUF = 4

_mesh = plsc.VectorSubcoreMesh(core_axis_name="c", subcore_axis_name="s")
_sc_params = pltpu.CompilerParams(use_tc_tiling_on_sc=False,
                                  needs_layout_passes=False)


def _full(v):
    return jnp.full((16,), v, jnp.int32)


def _zero_acc(acc, buf, sid, width):
    zero = jnp.zeros((16,), jnp.float32)
    nv = width // 16

    def zrow(r, _):
        for c in range(nv):
            buf[r, pl.ds(c * 16, 16)] = zero
        return 0
    lax.fori_loop(0, B, zrow, 0)
    base = sid * NROW
    for off in range(0, NROW, B):
        sz = min(B, NROW - off)
        pltpu.sync_copy(buf.at[pl.ds(0, sz)], acc.at[pl.ds(base + off, sz)])


def _sc_cnt_body(dst2, attr1, out, acc, buf, dstb, attrb):
    cid = lax.axis_index("c")
    sid = lax.axis_index("s")
    tid = cid * NS + sid
    zero = jnp.zeros((16,), jnp.float32)
    one = jnp.ones((16,), jnp.float32)
    iota = lax.iota(jnp.int32, 16)

    _zero_acc(acc, buf, sid, T)
    plsc.subcore_barrier()

    def superbatch(sb, _):
        pltpu.sync_copy(dst2.at[pl.ds(tid * TROWS + sb * NJ, NJ)], dstb)
        pltpu.sync_copy(attr1.at[pl.ds(tid * TILE_E + sb * SB, SB)], attrb)

        def sub(j, _):
            ebase = tid * TILE_E + sb * SB + j * B

            def zrow2(r, _):
                buf[r, :] = zero
                return 0
            lax.fori_loop(0, B, zrow2, 0)

            def grp(g, _):
                ev = g * 16 + iota
                attrv = plsc.load_gather(attrb, [j * B + ev])
                valid = (ebase + ev) < E
                plsc.store_scatter(buf, [ev, attrv], one, mask=valid)
                return 0
            lax.fori_loop(0, B // 16, grp, 0)
            pltpu.sync_copy(buf, acc.at[dstb.at[j]], add=True)
            return 0
        lax.fori_loop(0, NJ, sub, 0)
        return 0
    lax.fori_loop(0, NSB, superbatch, 0)
    plsc.subcore_barrier()

    @pl.when(sid == 0)
    def _():
        pltpu.sync_copy(acc, out.at[pl.ds(cid * N, N)])


_sc_cnt = functools.partial(
    pl.kernel,
    out_type=jax.ShapeDtypeStruct((NC * N, T), jnp.float32),
    mesh=_mesh,
    compiler_params=_sc_params,
    scratch_types=[
        pltpu.VMEM_SHARED((N, T), jnp.float32),
        pltpu.VMEM((B, T), jnp.float32),
        pltpu.VMEM((NJ, CH), jnp.int32),
        pltpu.VMEM((SB,), jnp.int32),
    ],
)(_sc_cnt_body)


def _sc_layer_body(h, xcat0, xcat1, aldt0, aldt1, tbl0, tbl1,
                   src2, dst2, attr1, out, *refs):
    acc = refs[0]
    bufs = refs[1:1 + NBUF]
    aldbs = refs[1 + NBUF:1 + 2 * NBUF]
    srcb, dstb, attrb, tblb = refs[1 + 2 * NBUF:5 + 2 * NBUF]
    semg = refs[5 + 2 * NBUF:5 + 3 * NBUF]
    sema = refs[5 + 3 * NBUF:5 + 4 * NBUF]
    sems = refs[5 + 4 * NBUF:5 + 5 * NBUF]
    hl = 4 if h == 8 else 1
    cid = lax.axis_index("c")
    sid = lax.axis_index("s")
    iota = lax.iota(jnp.int32, 16)
    hmask = iota < hl

    @pl.when(cid == 0)
    def _():
        pltpu.sync_copy(tbl0, tblb)

    @pl.when(cid == 1)
    def _():
        pltpu.sync_copy(tbl1, tblb)

    _zero_acc(acc, bufs[0], sid, W80)
    plsc.subcore_barrier()

    def superbatch(sb, _):
        row0 = sid * TROWS2 + sb * NJ
        pltpu.sync_copy(src2.at[pl.ds(row0, NJ)], srcb)
        pltpu.sync_copy(dst2.at[pl.ds(row0, NJ)], dstb)
        pltpu.sync_copy(attr1.at[pl.ds(sid * TILE_E2 + sb * SB, SB)], attrb)

        for half in range(2):
            for b in range(NBUF):
                j = half * NBUF + b

                def drain(b=b, j=j):
                    pltpu.make_async_copy(bufs[b], acc.at[dstb.at[j]],
                                          sems[b]).wait()
                if half == 0:
                    @pl.when(sb > 0)
                    def _():
                        drain()
                else:
                    drain()

                @pl.when(cid == 0)
                def _(b=b, j=j):
                    pltpu.async_copy(xcat0.at[srcb.at[j]], bufs[b], semg[b])
                    pltpu.async_copy(aldt0.at[dstb.at[j]], aldbs[b], sema[b])

                @pl.when(cid == 1)
                def _(b=b, j=j):
                    pltpu.async_copy(xcat1.at[srcb.at[j]], bufs[b], semg[b])
                    pltpu.async_copy(aldt1.at[dstb.at[j]], aldbs[b], sema[b])

            for b in range(NBUF):
                j = half * NBUF + b
                buf = bufs[b]
                aldb = aldbs[b]
                pltpu.make_async_copy(xcat0.at[srcb.at[j]], buf, semg[b]).wait()
                pltpu.make_async_copy(aldt0.at[dstb.at[j]], aldb, sema[b]).wait()
                ebase = sid * TILE_E2 + sb * SB + j * B

                def grp(g, _, buf=buf, aldb=aldb, j=j, ebase=ebase):
                    attrv16 = attrb[pl.ds(j * B + g * 16, 16)]
                    for ee in range(16):
                        e = g * 16 + ee
                        attr_e = attrv16[ee]
                        alsv = buf[e, pl.ds(64, 16)]
                        aldv = aldb[e, :]
                        tblv = tblb[pl.ds(attr_e * 16, 16)]
                        sv = alsv + aldv + tblv
                        sv = jnp.where(sv > 0, sv, 0.2 * sv)
                        ok = (ebase + e) < E
                        p = jnp.where(jnp.logical_and(hmask, ok),
                                      jnp.exp(sv), 0.0)
                        buf[e, pl.ds(64, 16)] = p
                        for cg in range(4):
                            ps = p[cg if h == 8 else 0]
                            buf[e, pl.ds(cg * 16, 16)] = \
                                buf[e, pl.ds(cg * 16, 16)] * ps
                    return 0
                lax.fori_loop(0, B // 16, grp, 0)
                pltpu.async_copy(buf, acc.at[dstb.at[j]], sems[b], add=True)
        return 0
    lax.fori_loop(0, NSB2, superbatch, 0)
    for b in range(NBUF):
        pltpu.make_async_copy(bufs[b], acc.at[dstb.at[NBUF + b]], sems[b]).wait()
    plsc.subcore_barrier()

    @pl.when(sid == 0)
    def _():
        pltpu.sync_copy(acc, out.at[pl.ds(cid * N, N)])


def _make_sc_layer(h):
    return functools.partial(
        pl.kernel,
        out_type=jax.ShapeDtypeStruct((NC * N, W80), jnp.float32),
        mesh=_mesh,
        compiler_params=_sc_params,
        scratch_types=[pltpu.VMEM_SHARED((N, W80), jnp.float32)]
        + [pltpu.VMEM((CH, W80), jnp.float32)] * NBUF
        + [pltpu.VMEM((CH, 16), jnp.float32)] * NBUF
        + [pltpu.VMEM((NJ, CH), jnp.int32),
           pltpu.VMEM((NJ, CH), jnp.int32),
           pltpu.VMEM((SB,), jnp.int32),
           pltpu.VMEM((T * 16,), jnp.float32)]
        + [pltpu.SemaphoreType.DMA] * (3 * NBUF),
    )(functools.partial(_sc_layer_body, h))


_sc_layer8 = _make_sc_layer(8)
_sc_layer1 = _make_sc_layer(1)


R = 2000
G = N // R


def _prologue_body(c0, c1, emb, *refs):
    was = refs[:8]
    loope = refs[8]
    touts = refs[9:]
    i = pl.program_id(0)
    cntT = c0[...] + c1[...]
    cnt = jnp.sum(cntT, axis=1, keepdims=True)
    loope[...] = jnp.dot(cntT, emb[...], preferred_element_type=jnp.float32) \
        / jnp.maximum(cnt, 1.0)

    @pl.when(i == 0)
    def _():
        e = emb[...]
        for k in range(8):
            touts[k][...] = jnp.dot(e, was[k][...],
                                    preferred_element_type=jnp.float32)


def _prologue(c0, c1, emb, was):
    fix = lambda i: (0, 0)
    blk = lambda i: (i, 0)
    return pl.pallas_call(
        _prologue_body,
        grid=(G,),
        in_specs=[pl.BlockSpec((R, T), blk), pl.BlockSpec((R, T), blk),
                  pl.BlockSpec((T, ED), fix)] +
                 [pl.BlockSpec((ED, 16), fix)] * 8,
        out_specs=[pl.BlockSpec((R, ED), blk)] + [pl.BlockSpec((T, 16), fix)] * 8,
        out_shape=[jax.ShapeDtypeStruct((N, ED), jnp.float32)] +
                  [jax.ShapeDtypeStruct((T, 16), jnp.float32)] * 8,
    )(c0, c1, emb, *was)


def _prep_body(x, wc0, wc1, wd0, wd1, xc0, xc1, al0, al1):
    xb = x[...]
    xc0[...] = jnp.dot(xb, wc0[...], preferred_element_type=jnp.float32)
    xc1[...] = jnp.dot(xb, wc1[...], preferred_element_type=jnp.float32)
    al0[...] = jnp.dot(xb, wd0[...], preferred_element_type=jnp.float32)
    al1[...] = jnp.dot(xb, wd1[...], preferred_element_type=jnp.float32)


def _prep(xin, Wc0, Wc1, Wd0, Wd1):
    din = xin.shape[1]
    fix = lambda i: (0, 0)
    blk = lambda i: (i, 0)
    return pl.pallas_call(
        _prep_body,
        grid=(G,),
        in_specs=[pl.BlockSpec((R, din), blk)] +
                 [pl.BlockSpec((din, W80), fix)] * 2 +
                 [pl.BlockSpec((din, 16), fix)] * 2,
        out_specs=[pl.BlockSpec((R, W80), blk)] * 2 +
                  [pl.BlockSpec((R, 16), blk)] * 2,
        out_shape=[jax.ShapeDtypeStruct((N, W80), jnp.float32)] * 2 +
                  [jax.ShapeDtypeStruct((N, 16), jnp.float32)] * 2,
    )(xin, Wc0, Wc1, Wd0, Wd1)


def _finalize_body(h, relu, w0, w1, pz0, pz1, xs0, xs1, als0, als1,
                   ald0, ald1, loope, wea0, wea1, bias, xprev, xout):
    hl = 4 if h == 8 else 1
    rep = 64 // hl
    le = loope[...]
    halves = []
    for wref, pzref, xsref, alsref, aldref, wearef in (
            (w0, pz0, xs0, als0, ald0, wea0),
            (w1, pz1, xs1, als1, ald1, wea1)):
        ae = jnp.dot(le, wearef[...], preferred_element_type=jnp.float32)
        sl = alsref[...] + aldref[...] + ae
        ploop = jnp.exp(jnp.where(sl > 0, sl, 0.2 * sl))
        zt = pzref[...] + ploop + 1e-16
        y = (wref[...] + xsref[...] * jnp.repeat(ploop[:, :hl], rep, axis=1)) \
            / jnp.repeat(zt[:, :hl], rep, axis=1)
        halves.append(y)
    y = jnp.concatenate(halves, axis=1) + bias[...]
    if relu:
        y = jnp.maximum(y, 0.0)
    xout[...] = y + xprev[...]


def _finalize(h, relu, w0, w1, pz0, pz1, xs0, xs1, als0, als1,
              ald0, ald1, loope, wea0, wea1, bias, xprev):
    fix = lambda i: (0, 0)
    blk = lambda i: (i, 0)
    return pl.pallas_call(
        functools.partial(_finalize_body, h, relu),
        grid=(G,),
        in_specs=[pl.BlockSpec((R, 64), blk)] * 2 +
                 [pl.BlockSpec((R, 16), blk)] * 2 +
                 [pl.BlockSpec((R, 64), blk)] * 2 +
                 [pl.BlockSpec((R, 16), blk)] * 4 +
                 [pl.BlockSpec((R, ED), blk)] +
                 [pl.BlockSpec((ED, 16), fix)] * 2 +
                 [pl.BlockSpec((1, 128), fix), pl.BlockSpec((R, 128), blk)],
        out_specs=pl.BlockSpec((R, 128), blk),
        out_shape=jax.ShapeDtypeStruct((N, 128), jnp.float32),
    )(w0, w1, pz0, pz1, xs0, xs1, als0, als1, ald0, ald1,
      loope, wea0, wea1, bias, xprev)


def _weight_prep(W, We, a_s, a_d, a_e, h, oc):
    din = W.shape[0]
    Wr = W.reshape(din, h, oc)
    ws = jnp.einsum('dho,ho->dh', Wr, a_s)
    wd = jnp.einsum('dho,ho->dh', Wr, a_d)
    wea = jnp.einsum('kho,ho->kh', We.reshape(ED, h, oc), a_e)
    hl = 4 if h == 8 else 1
    zc = jnp.zeros((din, 16 - hl), jnp.float32)
    ze = jnp.zeros((ED, 16 - hl), jnp.float32)
    if h == 8:
        Wc0 = jnp.concatenate([W[:, :64], ws[:, :4], zc], axis=1)
        Wc1 = jnp.concatenate([W[:, 64:], ws[:, 4:], zc], axis=1)
        Wd0 = jnp.concatenate([wd[:, :4], zc], axis=1)
        Wd1 = jnp.concatenate([wd[:, 4:], zc], axis=1)
        wea0 = jnp.concatenate([wea[:, :4], ze], axis=1)
        wea1 = jnp.concatenate([wea[:, 4:], ze], axis=1)
    else:
        Wc0 = jnp.concatenate([W[:, :64], ws, zc], axis=1)
        Wc1 = jnp.concatenate([W[:, 64:], ws, zc], axis=1)
        Wd0 = jnp.concatenate([wd, zc], axis=1)
        Wd1 = Wd0
        wea0 = jnp.concatenate([wea, ze], axis=1)
        wea1 = wea0
    return Wc0, Wc1, Wd0, Wd1, wea0, wea1


def kernel(x, edge_index, batch, edge_attr, edge_emb,
           W1, We1, as1, ad1, ae1, b1,
           W2, We2, as2, ad2, ae2, b2,
           W3, We3, as3, ad3, ae3, b3,
           W4, We4, as4, ad4, ae4, b4):
    src = edge_index[0]
    dst = edge_index[1]
    pad = EPAD - E
    src2 = jnp.pad(src, (0, pad)).reshape(EROWS, CH)
    dst2 = jnp.pad(dst, (0, pad)).reshape(EROWS, CH)
    attr1 = jnp.pad(edge_attr, (0, pad))

    layers = [
        _weight_prep(W1, We1, as1, ad1, ae1, 8, 16) + (b1, 8, True),
        _weight_prep(W2, We2, as2, ad2, ae2, 8, 16) + (b2, 8, True),
        _weight_prep(W3, We3, as3, ad3, ae3, 8, 16) + (b3, 8, True),
        _weight_prep(W4, We4, as4, ad4, ae4, 1, 128) + (b4, 1, False),
    ]

    cntp = _sc_cnt(dst2, attr1)
    was = [l[4] for l in layers] + [l[5] for l in layers]
    pro = _prologue(cntp[:N], cntp[N:], edge_emb, was)
    loope = pro[0]
    tbls0 = pro[1:5]
    tbls1 = pro[5:9]

    xcur = x
    for li, (Wc0, Wc1, Wd0, Wd1, wea0, wea1, b, h, relu) in enumerate(layers):
        resid = jnp.zeros((N, 128), jnp.float32) if li == 0 else xcur
        xc0, xc1, al0, al1 = _prep(xcur, Wc0, Wc1, Wd0, Wd1)
        accp = (_sc_layer8 if h == 8 else _sc_layer1)(
            xc0, xc1, al0, al1,
            tbls0[li].reshape(T * 16), tbls1[li].reshape(T * 16),
            src2, dst2, attr1)
        a0, a1 = accp[:N], accp[N:]
        xcur = _finalize(h, relu,
                         a0[:, :64], a1[:, :64], a0[:, 64:], a1[:, 64:],
                         xc0[:, :64], xc1[:, :64], xc0[:, 64:], xc1[:, 64:],
                         al0, al1, loope, wea0, wea1,
                         b.reshape(1, 128), resid)
    return (xcur, batch)


# speedup vs baseline: 35.2526x; 1.0070x over previous
import functools

import jax
import jax.numpy as jnp
import numpy as np
from jax import lax
from jax.experimental import pallas as pl
from jax.experimental.pallas import tpu as pltpu
from jax.experimental.pallas import tpu_sc as plsc

N = 10000
E = 320000
D = 128
ED = 32
T = 16

NC = 2
NS = 16
NT = NC * NS
CH = 128
B = 128
SB = 1024
NSB = 10
NJ = SB // B
TILE_E = NSB * SB
TROWS = TILE_E // CH
EPAD = NT * TILE_E
EROWS = EPAD // CH
NROW = N // NS

W80 = 80
TILE_E2 = EPAD // NS
TROWS2 = TILE_E2 // CH
NSB2 = TILE_E2 // SB
NBUF = 4

_mesh = plsc.VectorSubcoreMesh(core_axis_name="c", subcore_axis_name="s")
_sc_params = pltpu.CompilerParams(use_tc_tiling_on_sc=False,
                                  needs_layout_passes=False)


def _full(v):
    return jnp.full((16,), v, jnp.int32)


def _zero_acc(acc, buf, sid, width):
    zero = jnp.zeros((16,), jnp.float32)
    nv = width // 16

    def zrow(r, _):
        for c in range(nv):
            buf[r, pl.ds(c * 16, 16)] = zero
        return 0
    lax.fori_loop(0, B, zrow, 0)
    base = sid * NROW
    for off in range(0, NROW, B):
        sz = min(B, NROW - off)
        pltpu.sync_copy(buf.at[pl.ds(0, sz)], acc.at[pl.ds(base + off, sz)])


def _sc_cnt_body(dst2, attr1, out, acc, buf, dstb, attrb):
    cid = lax.axis_index("c")
    sid = lax.axis_index("s")
    tid = cid * NS + sid
    zero = jnp.zeros((16,), jnp.float32)
    one = jnp.ones((16,), jnp.float32)
    iota = lax.iota(jnp.int32, 16)

    _zero_acc(acc, buf, sid, T)
    plsc.subcore_barrier()

    def superbatch(sb, _):
        pltpu.sync_copy(dst2.at[pl.ds(tid * TROWS + sb * NJ, NJ)], dstb)
        pltpu.sync_copy(attr1.at[pl.ds(tid * TILE_E + sb * SB, SB)], attrb)

        def sub(j, _):
            ebase = tid * TILE_E + sb * SB + j * B

            def zrow2(r, _):
                buf[r, :] = zero
                return 0
            lax.fori_loop(0, B, zrow2, 0)

            def grp(g, _):
                ev = g * 16 + iota
                attrv = plsc.load_gather(attrb, [j * B + ev])
                valid = (ebase + ev) < E
                plsc.store_scatter(buf, [ev, attrv], one, mask=valid)
                return 0
            lax.fori_loop(0, B // 16, grp, 0)
            pltpu.sync_copy(buf, acc.at[dstb.at[j]], add=True)
            return 0
        lax.fori_loop(0, NJ, sub, 0)
        return 0
    lax.fori_loop(0, NSB, superbatch, 0)
    plsc.subcore_barrier()

    @pl.when(sid == 0)
    def _():
        pltpu.sync_copy(acc, out.at[pl.ds(cid * N, N)])


_sc_cnt = functools.partial(
    pl.kernel,
    out_type=jax.ShapeDtypeStruct((NC * N, T), jnp.float32),
    mesh=_mesh,
    compiler_params=_sc_params,
    scratch_types=[
        pltpu.VMEM_SHARED((N, T), jnp.float32),
        pltpu.VMEM((B, T), jnp.float32),
        pltpu.VMEM((NJ, CH), jnp.int32),
        pltpu.VMEM((SB,), jnp.int32),
    ],
)(_sc_cnt_body)


def _sc_layer_body(h, xcat0, xcat1, aldt0, aldt1, tbl0, tbl1,
                   src2, dst2, attr1, out, *refs):
    acc = refs[0]
    bufs = refs[1:1 + NBUF]
    aldbs = refs[1 + NBUF:1 + 2 * NBUF]
    srcb, dstb, attrb, tblb = refs[1 + 2 * NBUF:5 + 2 * NBUF]
    semg = refs[5 + 2 * NBUF:5 + 3 * NBUF]
    sema = refs[5 + 3 * NBUF:5 + 4 * NBUF]
    sems = refs[5 + 4 * NBUF:5 + 5 * NBUF]
    hl = 4 if h == 8 else 1
    cid = lax.axis_index("c")
    sid = lax.axis_index("s")
    iota = lax.iota(jnp.int32, 16)
    hmask = iota < hl

    @pl.when(cid == 0)
    def _():
        pltpu.sync_copy(tbl0, tblb)

    @pl.when(cid == 1)
    def _():
        pltpu.sync_copy(tbl1, tblb)

    _zero_acc(acc, bufs[0], sid, W80)
    plsc.subcore_barrier()

    def superbatch(sb, _):
        row0 = sid * TROWS2 + sb * NJ
        pltpu.sync_copy(src2.at[pl.ds(row0, NJ)], srcb)
        pltpu.sync_copy(dst2.at[pl.ds(row0, NJ)], dstb)
        pltpu.sync_copy(attr1.at[pl.ds(sid * TILE_E2 + sb * SB, SB)], attrb)

        for half in range(2):
            for b in range(NBUF):
                j = half * NBUF + b

                def drain(b=b, j=j):
                    pltpu.make_async_copy(bufs[b], acc.at[dstb.at[j]],
                                          sems[b]).wait()
                if half == 0:
                    @pl.when(sb > 0)
                    def _():
                        drain()
                else:
                    drain()

                @pl.when(cid == 0)
                def _(b=b, j=j):
                    pltpu.async_copy(xcat0.at[srcb.at[j]], bufs[b], semg[b])
                    pltpu.async_copy(aldt0.at[dstb.at[j]], aldbs[b], sema[b])

                @pl.when(cid == 1)
                def _(b=b, j=j):
                    pltpu.async_copy(xcat1.at[srcb.at[j]], bufs[b], semg[b])
                    pltpu.async_copy(aldt1.at[dstb.at[j]], aldbs[b], sema[b])

            for b in range(NBUF):
                j = half * NBUF + b
                buf = bufs[b]
                aldb = aldbs[b]
                pltpu.make_async_copy(xcat0.at[srcb.at[j]], buf, semg[b]).wait()
                pltpu.make_async_copy(aldt0.at[dstb.at[j]], aldb, sema[b]).wait()
                ebase = sid * TILE_E2 + sb * SB + j * B

                def grp(g, _, buf=buf, aldb=aldb, j=j, ebase=ebase):
                    attrv16 = attrb[pl.ds(j * B + g * 16, 16)]
                    for ee in range(16):
                        e = g * 16 + ee
                        attr_b = jnp.take(attrv16, _full(ee))
                        alsv = buf[e, pl.ds(64, 16)]
                        aldv = aldb[e, :]
                        tblv = plsc.load_gather(tblb, [attr_b * 16 + iota])
                        sv = alsv + aldv + tblv
                        sv = jnp.where(sv > 0, sv, 0.2 * sv)
                        ok = (ebase + e) < E
                        p = jnp.where(jnp.logical_and(hmask, ok),
                                      jnp.exp(sv), 0.0)
                        buf[e, pl.ds(64, 16)] = p
                        for cg in range(4):
                            ps = jnp.take(p, _full(cg if h == 8 else 0))
                            buf[e, pl.ds(cg * 16, 16)] = \
                                buf[e, pl.ds(cg * 16, 16)] * ps
                    return 0
                lax.fori_loop(0, B // 16, grp, 0)
                pltpu.async_copy(buf, acc.at[dstb.at[j]], sems[b], add=True)
        return 0
    lax.fori_loop(0, NSB2, superbatch, 0)
    for b in range(NBUF):
        pltpu.make_async_copy(bufs[b], acc.at[dstb.at[NBUF + b]], sems[b]).wait()
    plsc.subcore_barrier()

    @pl.when(sid == 0)
    def _():
        pltpu.sync_copy(acc, out.at[pl.ds(cid * N, N)])


def _make_sc_layer(h):
    return functools.partial(
        pl.kernel,
        out_type=jax.ShapeDtypeStruct((NC * N, W80), jnp.float32),
        mesh=_mesh,
        compiler_params=_sc_params,
        scratch_types=[pltpu.VMEM_SHARED((N, W80), jnp.float32)]
        + [pltpu.VMEM((CH, W80), jnp.float32)] * NBUF
        + [pltpu.VMEM((CH, 16), jnp.float32)] * NBUF
        + [pltpu.VMEM((NJ, CH), jnp.int32),
           pltpu.VMEM((NJ, CH), jnp.int32),
           pltpu.VMEM((SB,), jnp.int32),
           pltpu.VMEM((T * 16,), jnp.float32)]
        + [pltpu.SemaphoreType.DMA] * (3 * NBUF),
    )(functools.partial(_sc_layer_body, h))


_sc_layer8 = _make_sc_layer(8)
_sc_layer1 = _make_sc_layer(1)


R = 2000
G = N // R


def _prologue_body(c0, c1, emb, *refs):
    was = refs[:8]
    loope = refs[8]
    touts = refs[9:]
    i = pl.program_id(0)
    cntT = c0[...] + c1[...]
    cnt = jnp.sum(cntT, axis=1, keepdims=True)
    loope[...] = jnp.dot(cntT, emb[...], preferred_element_type=jnp.float32) \
        / jnp.maximum(cnt, 1.0)

    @pl.when(i == 0)
    def _():
        e = emb[...]
        for k in range(8):
            touts[k][...] = jnp.dot(e, was[k][...],
                                    preferred_element_type=jnp.float32)


def _prologue(c0, c1, emb, was):
    fix = lambda i: (0, 0)
    blk = lambda i: (i, 0)
    return pl.pallas_call(
        _prologue_body,
        grid=(G,),
        in_specs=[pl.BlockSpec((R, T), blk), pl.BlockSpec((R, T), blk),
                  pl.BlockSpec((T, ED), fix)] +
                 [pl.BlockSpec((ED, 16), fix)] * 8,
        out_specs=[pl.BlockSpec((R, ED), blk)] + [pl.BlockSpec((T, 16), fix)] * 8,
        out_shape=[jax.ShapeDtypeStruct((N, ED), jnp.float32)] +
                  [jax.ShapeDtypeStruct((T, 16), jnp.float32)] * 8,
    )(c0, c1, emb, *was)


def _prep_body(x, wc0, wc1, wd0, wd1, xc0, xc1, al0, al1):
    xb = x[...]
    xc0[...] = jnp.dot(xb, wc0[...], preferred_element_type=jnp.float32)
    xc1[...] = jnp.dot(xb, wc1[...], preferred_element_type=jnp.float32)
    al0[...] = jnp.dot(xb, wd0[...], preferred_element_type=jnp.float32)
    al1[...] = jnp.dot(xb, wd1[...], preferred_element_type=jnp.float32)


def _prep(xin, Wc0, Wc1, Wd0, Wd1):
    din = xin.shape[1]
    fix = lambda i: (0, 0)
    blk = lambda i: (i, 0)
    return pl.pallas_call(
        _prep_body,
        grid=(G,),
        in_specs=[pl.BlockSpec((R, din), blk)] +
                 [pl.BlockSpec((din, W80), fix)] * 2 +
                 [pl.BlockSpec((din, 16), fix)] * 2,
        out_specs=[pl.BlockSpec((R, W80), blk)] * 2 +
                  [pl.BlockSpec((R, 16), blk)] * 2,
        out_shape=[jax.ShapeDtypeStruct((N, W80), jnp.float32)] * 2 +
                  [jax.ShapeDtypeStruct((N, 16), jnp.float32)] * 2,
    )(xin, Wc0, Wc1, Wd0, Wd1)


def _finalize_body(h, relu, w0, w1, pz0, pz1, xs0, xs1, als0, als1,
                   ald0, ald1, loope, wea0, wea1, bias, xprev, xout):
    hl = 4 if h == 8 else 1
    rep = 64 // hl
    le = loope[...]
    halves = []
    for wref, pzref, xsref, alsref, aldref, wearef in (
            (w0, pz0, xs0, als0, ald0, wea0),
            (w1, pz1, xs1, als1, ald1, wea1)):
        ae = jnp.dot(le, wearef[...], preferred_element_type=jnp.float32)
        sl = alsref[...] + aldref[...] + ae
        ploop = jnp.exp(jnp.where(sl > 0, sl, 0.2 * sl))
        zt = pzref[...] + ploop + 1e-16
        y = (wref[...] + xsref[...] * jnp.repeat(ploop[:, :hl], rep, axis=1)) \
            / jnp.repeat(zt[:, :hl], rep, axis=1)
        halves.append(y)
    y = jnp.concatenate(halves, axis=1) + bias[...]
    if relu:
        y = jnp.maximum(y, 0.0)
    xout[...] = y + xprev[...]


def _finalize(h, relu, w0, w1, pz0, pz1, xs0, xs1, als0, als1,
              ald0, ald1, loope, wea0, wea1, bias, xprev):
    fix = lambda i: (0, 0)
    blk = lambda i: (i, 0)
    return pl.pallas_call(
        functools.partial(_finalize_body, h, relu),
        grid=(G,),
        in_specs=[pl.BlockSpec((R, 64), blk)] * 2 +
                 [pl.BlockSpec((R, 16), blk)] * 2 +
                 [pl.BlockSpec((R, 64), blk)] * 2 +
                 [pl.BlockSpec((R, 16), blk)] * 4 +
                 [pl.BlockSpec((R, ED), blk)] +
                 [pl.BlockSpec((ED, 16), fix)] * 2 +
                 [pl.BlockSpec((1, 128), fix), pl.BlockSpec((R, 128), blk)],
        out_specs=pl.BlockSpec((R, 128), blk),
        out_shape=jax.ShapeDtypeStruct((N, 128), jnp.float32),
    )(w0, w1, pz0, pz1, xs0, xs1, als0, als1, ald0, ald1,
      loope, wea0, wea1, bias, xprev)


def _weight_prep(W, We, a_s, a_d, a_e, h, oc):
    din = W.shape[0]
    Wr = W.reshape(din, h, oc)
    ws = jnp.einsum('dho,ho->dh', Wr, a_s)
    wd = jnp.einsum('dho,ho->dh', Wr, a_d)
    wea = jnp.einsum('kho,ho->kh', We.reshape(ED, h, oc), a_e)
    hl = 4 if h == 8 else 1
    zc = jnp.zeros((din, 16 - hl), jnp.float32)
    ze = jnp.zeros((ED, 16 - hl), jnp.float32)
    if h == 8:
        Wc0 = jnp.concatenate([W[:, :64], ws[:, :4], zc], axis=1)
        Wc1 = jnp.concatenate([W[:, 64:], ws[:, 4:], zc], axis=1)
        Wd0 = jnp.concatenate([wd[:, :4], zc], axis=1)
        Wd1 = jnp.concatenate([wd[:, 4:], zc], axis=1)
        wea0 = jnp.concatenate([wea[:, :4], ze], axis=1)
        wea1 = jnp.concatenate([wea[:, 4:], ze], axis=1)
    else:
        Wc0 = jnp.concatenate([W[:, :64], ws, zc], axis=1)
        Wc1 = jnp.concatenate([W[:, 64:], ws, zc], axis=1)
        Wd0 = jnp.concatenate([wd, zc], axis=1)
        Wd1 = Wd0
        wea0 = jnp.concatenate([wea, ze], axis=1)
        wea1 = wea0
    return Wc0, Wc1, Wd0, Wd1, wea0, wea1


def kernel(x, edge_index, batch, edge_attr, edge_emb,
           W1, We1, as1, ad1, ae1, b1,
           W2, We2, as2, ad2, ae2, b2,
           W3, We3, as3, ad3, ae3, b3,
           W4, We4, as4, ad4, ae4, b4):
    src = edge_index[0]
    dst = edge_index[1]
    pad = EPAD - E
    src2 = jnp.pad(src, (0, pad)).reshape(EROWS, CH)
    dst2 = jnp.pad(dst, (0, pad)).reshape(EROWS, CH)
    attr1 = jnp.pad(edge_attr, (0, pad))

    layers = [
        _weight_prep(W1, We1, as1, ad1, ae1, 8, 16) + (b1, 8, True),
        _weight_prep(W2, We2, as2, ad2, ae2, 8, 16) + (b2, 8, True),
        _weight_prep(W3, We3, as3, ad3, ae3, 8, 16) + (b3, 8, True),
        _weight_prep(W4, We4, as4, ad4, ae4, 1, 128) + (b4, 1, False),
    ]

    cntp = _sc_cnt(dst2, attr1)
    was = [l[4] for l in layers] + [l[5] for l in layers]
    pro = _prologue(cntp[:N], cntp[N:], edge_emb, was)
    loope = pro[0]
    tbls0 = pro[1:5]
    tbls1 = pro[5:9]

    xcur = x
    for li, (Wc0, Wc1, Wd0, Wd1, wea0, wea1, b, h, relu) in enumerate(layers):
        resid = jnp.zeros((N, 128), jnp.float32) if li == 0 else xcur
        xc0, xc1, al0, al1 = _prep(xcur, Wc0, Wc1, Wd0, Wd1)
        accp = (_sc_layer8 if h == 8 else _sc_layer1)(
            xc0, xc1, al0, al1,
            tbls0[li].reshape(T * 16), tbls1[li].reshape(T * 16),
            src2, dst2, attr1)
        a0, a1 = accp[:N], accp[N:]
        xcur = _finalize(h, relu,
                         a0[:, :64], a1[:, :64], a0[:, 64:], a1[:, 64:],
                         xc0[:, :64], xc1[:, :64], xc0[:, 64:], xc1[:, 64:],
                         al0, al1, loope, wea0, wea1,
                         b.reshape(1, 128), resid)
    return (xcur, batch)


# speedup vs baseline: 42.9230x; 1.2176x over previous
import functools

import jax
import jax.numpy as jnp
import numpy as np
from jax import lax
from jax.experimental import pallas as pl
from jax.experimental.pallas import tpu as pltpu
from jax.experimental.pallas import tpu_sc as plsc

N = 10000
E = 320000
D = 128
ED = 32
T = 16

NC = 2
NS = 16
NT = NC * NS
CH = 128
B = 128
SB = 1024
NSB = 10
NJ = SB // B
TILE_E = NSB * SB
TROWS = TILE_E // CH
EPAD = NT * TILE_E
EROWS = EPAD // CH
NROW = N // NS

W80 = 80
TILE_E2 = EPAD // NS
TROWS2 = TILE_E2 // CH
NSB2 = TILE_E2 // SB
NBUF = 4

_mesh = plsc.VectorSubcoreMesh(core_axis_name="c", subcore_axis_name="s")
_sc_params = pltpu.CompilerParams(use_tc_tiling_on_sc=False,
                                  needs_layout_passes=False)


def _full(v):
    return jnp.full((16,), v, jnp.int32)


def _zero_acc(acc, buf, sid, width):
    zero = jnp.zeros((16,), jnp.float32)
    nv = width // 16

    def zrow(r, _):
        for c in range(nv):
            buf[r, pl.ds(c * 16, 16)] = zero
        return 0
    lax.fori_loop(0, B, zrow, 0)
    base = sid * NROW
    for off in range(0, NROW, B):
        sz = min(B, NROW - off)
        pltpu.sync_copy(buf.at[pl.ds(0, sz)], acc.at[pl.ds(base + off, sz)])


def _sc_cnt_body(dst2, attr1, out, acc, buf, dstb, attrb):
    cid = lax.axis_index("c")
    sid = lax.axis_index("s")
    tid = cid * NS + sid
    zero = jnp.zeros((16,), jnp.float32)
    one = jnp.ones((16,), jnp.float32)
    iota = lax.iota(jnp.int32, 16)

    _zero_acc(acc, buf, sid, T)
    plsc.subcore_barrier()

    def superbatch(sb, _):
        pltpu.sync_copy(dst2.at[pl.ds(tid * TROWS + sb * NJ, NJ)], dstb)
        pltpu.sync_copy(attr1.at[pl.ds(tid * TILE_E + sb * SB, SB)], attrb)

        def sub(j, _):
            ebase = tid * TILE_E + sb * SB + j * B

            def zrow2(r, _):
                buf[r, :] = zero
                return 0
            lax.fori_loop(0, B, zrow2, 0)

            def grp(g, _):
                ev = g * 16 + iota
                attrv = plsc.load_gather(attrb, [j * B + ev])
                valid = (ebase + ev) < E
                plsc.store_scatter(buf, [ev, attrv], one, mask=valid)
                return 0
            lax.fori_loop(0, B // 16, grp, 0)
            pltpu.sync_copy(buf, acc.at[dstb.at[j]], add=True)
            return 0
        lax.fori_loop(0, NJ, sub, 0)
        return 0
    lax.fori_loop(0, NSB, superbatch, 0)
    plsc.subcore_barrier()

    @pl.when(sid == 0)
    def _():
        pltpu.sync_copy(acc, out.at[pl.ds(cid * N, N)])


_sc_cnt = functools.partial(
    pl.kernel,
    out_type=jax.ShapeDtypeStruct((NC * N, T), jnp.float32),
    mesh=_mesh,
    compiler_params=_sc_params,
    scratch_types=[
        pltpu.VMEM_SHARED((N, T), jnp.float32),
        pltpu.VMEM((B, T), jnp.float32),
        pltpu.VMEM((NJ, CH), jnp.int32),
        pltpu.VMEM((SB,), jnp.int32),
    ],
)(_sc_cnt_body)


def _sc_layer_body(h, xcat0, xcat1, aldt0, aldt1, tbl0, tbl1,
                   src2, dst2, attr1, out, *refs):
    acc = refs[0]
    bufs = refs[1:1 + NBUF]
    aldbs = refs[1 + NBUF:1 + 2 * NBUF]
    srcb, dstb, attrb, tblb = refs[1 + 2 * NBUF:5 + 2 * NBUF]
    semg = refs[5 + 2 * NBUF:5 + 3 * NBUF]
    sema = refs[5 + 3 * NBUF:5 + 4 * NBUF]
    sems = refs[5 + 4 * NBUF:5 + 5 * NBUF]
    hl = 4 if h == 8 else 1
    cid = lax.axis_index("c")
    sid = lax.axis_index("s")
    iota = lax.iota(jnp.int32, 16)
    hmask = iota < hl

    @pl.when(cid == 0)
    def _():
        pltpu.sync_copy(tbl0, tblb)

    @pl.when(cid == 1)
    def _():
        pltpu.sync_copy(tbl1, tblb)

    _zero_acc(acc, bufs[0], sid, W80)
    plsc.subcore_barrier()

    def superbatch(sb, _):
        row0 = sid * TROWS2 + sb * NJ
        pltpu.sync_copy(src2.at[pl.ds(row0, NJ)], srcb)
        pltpu.sync_copy(dst2.at[pl.ds(row0, NJ)], dstb)
        pltpu.sync_copy(attr1.at[pl.ds(sid * TILE_E2 + sb * SB, SB)], attrb)

        for half in range(2):
            for b in range(NBUF):
                j = half * NBUF + b

                def drain(b=b, j=j):
                    pltpu.make_async_copy(bufs[b], acc.at[dstb.at[j]],
                                          sems[b]).wait()
                if half == 0:
                    @pl.when(sb > 0)
                    def _():
                        drain()
                else:
                    drain()

                @pl.when(cid == 0)
                def _(b=b, j=j):
                    pltpu.async_copy(xcat0.at[srcb.at[j]], bufs[b], semg[b])
                    pltpu.async_copy(aldt0.at[dstb.at[j]], aldbs[b], sema[b])

                @pl.when(cid == 1)
                def _(b=b, j=j):
                    pltpu.async_copy(xcat1.at[srcb.at[j]], bufs[b], semg[b])
                    pltpu.async_copy(aldt1.at[dstb.at[j]], aldbs[b], sema[b])

            for b in range(NBUF):
                j = half * NBUF + b
                buf = bufs[b]
                aldb = aldbs[b]
                pltpu.make_async_copy(xcat0.at[srcb.at[j]], buf, semg[b]).wait()
                pltpu.make_async_copy(aldt0.at[dstb.at[j]], aldb, sema[b]).wait()
                ebase = sid * TILE_E2 + sb * SB + j * B

                @plsc.parallel_loop(0, B // 16, 1, unroll=2)
                def _(g, buf=buf, aldb=aldb, j=j, ebase=ebase):
                    attrv16 = attrb[pl.ds(j * B + g * 16, 16)]
                    for ee in range(16):
                        e = g * 16 + ee
                        attr_b = jnp.take(attrv16, _full(ee))
                        alsv = buf[e, pl.ds(64, 16)]
                        aldv = aldb[e, :]
                        tblv = plsc.load_gather(tblb, [attr_b * 16 + iota])
                        sv = alsv + aldv + tblv
                        sv = jnp.where(sv > 0, sv, 0.2 * sv)
                        ok = (ebase + e) < E
                        p = jnp.where(jnp.logical_and(hmask, ok),
                                      jnp.exp(sv), 0.0)
                        buf[e, pl.ds(64, 16)] = p
                        for cg in range(4):
                            ps = jnp.take(p, _full(cg if h == 8 else 0))
                            buf[e, pl.ds(cg * 16, 16)] = \
                                buf[e, pl.ds(cg * 16, 16)] * ps
                pltpu.async_copy(buf, acc.at[dstb.at[j]], sems[b], add=True)
        return 0
    lax.fori_loop(0, NSB2, superbatch, 0)
    for b in range(NBUF):
        pltpu.make_async_copy(bufs[b], acc.at[dstb.at[NBUF + b]], sems[b]).wait()
    plsc.subcore_barrier()

    @pl.when(sid == 0)
    def _():
        pltpu.sync_copy(acc, out.at[pl.ds(cid * N, N)])


def _make_sc_layer(h):
    return functools.partial(
        pl.kernel,
        out_type=jax.ShapeDtypeStruct((NC * N, W80), jnp.float32),
        mesh=_mesh,
        compiler_params=_sc_params,
        scratch_types=[pltpu.VMEM_SHARED((N, W80), jnp.float32)]
        + [pltpu.VMEM((CH, W80), jnp.float32)] * NBUF
        + [pltpu.VMEM((CH, 16), jnp.float32)] * NBUF
        + [pltpu.VMEM((NJ, CH), jnp.int32),
           pltpu.VMEM((NJ, CH), jnp.int32),
           pltpu.VMEM((SB,), jnp.int32),
           pltpu.VMEM((T * 16,), jnp.float32)]
        + [pltpu.SemaphoreType.DMA] * (3 * NBUF),
    )(functools.partial(_sc_layer_body, h))


_sc_layer8 = _make_sc_layer(8)
_sc_layer1 = _make_sc_layer(1)


R = 2000
G = N // R


def _prologue_body(c0, c1, emb, *refs):
    was = refs[:8]
    loope = refs[8]
    touts = refs[9:]
    i = pl.program_id(0)
    cntT = c0[...] + c1[...]
    cnt = jnp.sum(cntT, axis=1, keepdims=True)
    loope[...] = jnp.dot(cntT, emb[...], preferred_element_type=jnp.float32) \
        / jnp.maximum(cnt, 1.0)

    @pl.when(i == 0)
    def _():
        e = emb[...]
        for k in range(8):
            touts[k][...] = jnp.dot(e, was[k][...],
                                    preferred_element_type=jnp.float32)


def _prologue(c0, c1, emb, was):
    fix = lambda i: (0, 0)
    blk = lambda i: (i, 0)
    return pl.pallas_call(
        _prologue_body,
        grid=(G,),
        in_specs=[pl.BlockSpec((R, T), blk), pl.BlockSpec((R, T), blk),
                  pl.BlockSpec((T, ED), fix)] +
                 [pl.BlockSpec((ED, 16), fix)] * 8,
        out_specs=[pl.BlockSpec((R, ED), blk)] + [pl.BlockSpec((T, 16), fix)] * 8,
        out_shape=[jax.ShapeDtypeStruct((N, ED), jnp.float32)] +
                  [jax.ShapeDtypeStruct((T, 16), jnp.float32)] * 8,
    )(c0, c1, emb, *was)


def _prep_body(x, wc0, wc1, wd0, wd1, xc0, xc1, al0, al1):
    xb = x[...]
    xc0[...] = jnp.dot(xb, wc0[...], preferred_element_type=jnp.float32)
    xc1[...] = jnp.dot(xb, wc1[...], preferred_element_type=jnp.float32)
    al0[...] = jnp.dot(xb, wd0[...], preferred_element_type=jnp.float32)
    al1[...] = jnp.dot(xb, wd1[...], preferred_element_type=jnp.float32)


def _prep(xin, Wc0, Wc1, Wd0, Wd1):
    din = xin.shape[1]
    fix = lambda i: (0, 0)
    blk = lambda i: (i, 0)
    return pl.pallas_call(
        _prep_body,
        grid=(G,),
        in_specs=[pl.BlockSpec((R, din), blk)] +
                 [pl.BlockSpec((din, W80), fix)] * 2 +
                 [pl.BlockSpec((din, 16), fix)] * 2,
        out_specs=[pl.BlockSpec((R, W80), blk)] * 2 +
                  [pl.BlockSpec((R, 16), blk)] * 2,
        out_shape=[jax.ShapeDtypeStruct((N, W80), jnp.float32)] * 2 +
                  [jax.ShapeDtypeStruct((N, 16), jnp.float32)] * 2,
    )(xin, Wc0, Wc1, Wd0, Wd1)


def _finalize_body(h, relu, w0, w1, pz0, pz1, xs0, xs1, als0, als1,
                   ald0, ald1, loope, wea0, wea1, bias, xprev, xout):
    hl = 4 if h == 8 else 1
    rep = 64 // hl
    le = loope[...]
    halves = []
    for wref, pzref, xsref, alsref, aldref, wearef in (
            (w0, pz0, xs0, als0, ald0, wea0),
            (w1, pz1, xs1, als1, ald1, wea1)):
        ae = jnp.dot(le, wearef[...], preferred_element_type=jnp.float32)
        sl = alsref[...] + aldref[...] + ae
        ploop = jnp.exp(jnp.where(sl > 0, sl, 0.2 * sl))
        zt = pzref[...] + ploop + 1e-16
        y = (wref[...] + xsref[...] * jnp.repeat(ploop[:, :hl], rep, axis=1)) \
            / jnp.repeat(zt[:, :hl], rep, axis=1)
        halves.append(y)
    y = jnp.concatenate(halves, axis=1) + bias[...]
    if relu:
        y = jnp.maximum(y, 0.0)
    xout[...] = y + xprev[...]


def _finalize(h, relu, w0, w1, pz0, pz1, xs0, xs1, als0, als1,
              ald0, ald1, loope, wea0, wea1, bias, xprev):
    fix = lambda i: (0, 0)
    blk = lambda i: (i, 0)
    return pl.pallas_call(
        functools.partial(_finalize_body, h, relu),
        grid=(G,),
        in_specs=[pl.BlockSpec((R, 64), blk)] * 2 +
                 [pl.BlockSpec((R, 16), blk)] * 2 +
                 [pl.BlockSpec((R, 64), blk)] * 2 +
                 [pl.BlockSpec((R, 16), blk)] * 4 +
                 [pl.BlockSpec((R, ED), blk)] +
                 [pl.BlockSpec((ED, 16), fix)] * 2 +
                 [pl.BlockSpec((1, 128), fix), pl.BlockSpec((R, 128), blk)],
        out_specs=pl.BlockSpec((R, 128), blk),
        out_shape=jax.ShapeDtypeStruct((N, 128), jnp.float32),
    )(w0, w1, pz0, pz1, xs0, xs1, als0, als1, ald0, ald1,
      loope, wea0, wea1, bias, xprev)


def _weight_prep(W, We, a_s, a_d, a_e, h, oc):
    din = W.shape[0]
    Wr = W.reshape(din, h, oc)
    ws = jnp.einsum('dho,ho->dh', Wr, a_s)
    wd = jnp.einsum('dho,ho->dh', Wr, a_d)
    wea = jnp.einsum('kho,ho->kh', We.reshape(ED, h, oc), a_e)
    hl = 4 if h == 8 else 1
    zc = jnp.zeros((din, 16 - hl), jnp.float32)
    ze = jnp.zeros((ED, 16 - hl), jnp.float32)
    if h == 8:
        Wc0 = jnp.concatenate([W[:, :64], ws[:, :4], zc], axis=1)
        Wc1 = jnp.concatenate([W[:, 64:], ws[:, 4:], zc], axis=1)
        Wd0 = jnp.concatenate([wd[:, :4], zc], axis=1)
        Wd1 = jnp.concatenate([wd[:, 4:], zc], axis=1)
        wea0 = jnp.concatenate([wea[:, :4], ze], axis=1)
        wea1 = jnp.concatenate([wea[:, 4:], ze], axis=1)
    else:
        Wc0 = jnp.concatenate([W[:, :64], ws, zc], axis=1)
        Wc1 = jnp.concatenate([W[:, 64:], ws, zc], axis=1)
        Wd0 = jnp.concatenate([wd, zc], axis=1)
        Wd1 = Wd0
        wea0 = jnp.concatenate([wea, ze], axis=1)
        wea1 = wea0
    return Wc0, Wc1, Wd0, Wd1, wea0, wea1


def kernel(x, edge_index, batch, edge_attr, edge_emb,
           W1, We1, as1, ad1, ae1, b1,
           W2, We2, as2, ad2, ae2, b2,
           W3, We3, as3, ad3, ae3, b3,
           W4, We4, as4, ad4, ae4, b4):
    src = edge_index[0]
    dst = edge_index[1]
    pad = EPAD - E
    src2 = jnp.pad(src, (0, pad)).reshape(EROWS, CH)
    dst2 = jnp.pad(dst, (0, pad)).reshape(EROWS, CH)
    attr1 = jnp.pad(edge_attr, (0, pad))

    layers = [
        _weight_prep(W1, We1, as1, ad1, ae1, 8, 16) + (b1, 8, True),
        _weight_prep(W2, We2, as2, ad2, ae2, 8, 16) + (b2, 8, True),
        _weight_prep(W3, We3, as3, ad3, ae3, 8, 16) + (b3, 8, True),
        _weight_prep(W4, We4, as4, ad4, ae4, 1, 128) + (b4, 1, False),
    ]

    cntp = _sc_cnt(dst2, attr1)
    was = [l[4] for l in layers] + [l[5] for l in layers]
    pro = _prologue(cntp[:N], cntp[N:], edge_emb, was)
    loope = pro[0]
    tbls0 = pro[1:5]
    tbls1 = pro[5:9]

    xcur = x
    for li, (Wc0, Wc1, Wd0, Wd1, wea0, wea1, b, h, relu) in enumerate(layers):
        resid = jnp.zeros((N, 128), jnp.float32) if li == 0 else xcur
        xc0, xc1, al0, al1 = _prep(xcur, Wc0, Wc1, Wd0, Wd1)
        accp = (_sc_layer8 if h == 8 else _sc_layer1)(
            xc0, xc1, al0, al1,
            tbls0[li].reshape(T * 16), tbls1[li].reshape(T * 16),
            src2, dst2, attr1)
        a0, a1 = accp[:N], accp[N:]
        xcur = _finalize(h, relu,
                         a0[:, :64], a1[:, :64], a0[:, 64:], a1[:, 64:],
                         xc0[:, :64], xc1[:, :64], xc0[:, 64:], xc1[:, 64:],
                         al0, al1, loope, wea0, wea1,
                         b.reshape(1, 128), resid)
    return (xcur, batch)
